# baseline (speedup 1.0000x reference)
import functools

import jax
import jax.numpy as jnp
from jax import lax
from jax.experimental import pallas as pl
from jax.experimental.pallas import tpu as pltpu
from jax.experimental.pallas import tpu_sc as plsc

LAMBDA_DIFFUSION = 1.0
LAMBDA_CONTENT = 2.0

_B = 32
_N_PTS = 16384
_DIM = 3
_C = 512
_F = 1024

_N_DIFF = _B * _N_PTS * _DIM
_N_CONTENT = _B * _C * _F
_N_ROWS = _B * _C

_R_SC = 4096
_R_TC = _N_ROWS - _R_SC

_GRID = 16
_CONT_ROWS = _R_TC // _GRID
_PRED_B = _B // _GRID

_NW = 32
_ROWS_PER_W = _R_SC // _NW
_CHUNK = 32
_N_CHUNKS = _ROWS_PER_W // _CHUNK
_VECS = _F // 16


def _tc_kernel(pn_ref, tn_ref, co_ref, cf_ref, out_ref, acc_ref):
    i = pl.program_id(0)

    @pl.when(i == 0)
    def _init():
        for k in range(5):
            acc_ref[k] = 0.0

    d = pn_ref[...] - tn_ref[...]
    s_diff = jnp.sum(d * d)

    co = co_ref[...]
    cf = cf_ref[...]
    m = cf - co
    s_mse = jnp.sum(m * m)
    s_abs = jnp.sum(jnp.abs(co))

    inv_f = 1.0 / _F
    inv_fm1 = 1.0 / (_F - 1)
    sco = jnp.sum(co, axis=-1)
    sco2 = jnp.sum(co * co, axis=-1)
    s_var_o = jnp.sum((sco2 - sco * sco * inv_f) * inv_fm1)

    scf = jnp.sum(cf, axis=-1)
    scf2 = jnp.sum(cf * cf, axis=-1)
    s_var_n = jnp.sum((scf2 - scf * scf * inv_f) * inv_fm1)

    acc_ref[0] += s_diff
    acc_ref[1] += s_mse
    acc_ref[2] += s_var_o
    acc_ref[3] += s_var_n
    acc_ref[4] += s_abs

    @pl.when(i == _GRID - 1)
    def _finish():
        for k in range(5):
            out_ref[k] = acc_ref[k]


_sc_mesh = plsc.VectorSubcoreMesh(
    core_axis_name="c", subcore_axis_name="s", num_cores=2, num_subcores=16)


_GROUPS = _CHUNK // 16


@functools.partial(
    pl.kernel,
    out_type=jax.ShapeDtypeStruct((_NW, 4, 16), jnp.float32),
    mesh=_sc_mesh,
    scratch_types=[
        pltpu.VMEM((_CHUNK, _F), jnp.float32),
        pltpu.VMEM((_CHUNK, _F), jnp.float32),
        pltpu.VMEM((4, 16), jnp.float32),
    ],
    compiler_params=pltpu.CompilerParams(use_tc_tiling_on_sc=False, needs_layout_passes=False),
)
def _sc_content(co_hbm, cf_hbm, out_hbm, co_v, cf_v, out_v):
    wid = lax.axis_index("s") * 2 + lax.axis_index("c")
    base = _R_TC + wid * _ROWS_PER_W
    zero = jnp.zeros((16,), jnp.float32)
    lane = lax.iota(jnp.int32, 16)
    inv_f = 1.0 / _F
    inv_fm1 = 1.0 / (_F - 1)

    def chunk_body(j, carry):
        var_o, var_n, m_v, a_v = carry
        r0 = base + j * _CHUNK
        pltpu.sync_copy(co_hbm.at[pl.ds(r0, _CHUNK), :], co_v)
        pltpu.sync_copy(cf_hbm.at[pl.ds(r0, _CHUNK), :], cf_v)

        def group_body(g, carry2):
            var_o, var_n, m_v, a_v = carry2
            rows = g * 16 + lane

            def col_body(c, c3):
                s_o, q_o, s_n, q_n, m_v, a_v = c3
                cols = jnp.full((16,), c, jnp.int32)
                x = plsc.load_gather(co_v, [rows, cols])
                y = plsc.load_gather(cf_v, [rows, cols])
                s_o = s_o + x
                q_o = q_o + x * x
                s_n = s_n + y
                q_n = q_n + y * y
                d = y - x
                m_v = m_v + d * d
                a_v = a_v + jnp.abs(x)
                return (s_o, q_o, s_n, q_n, m_v, a_v)

            s_o, q_o, s_n, q_n, m_v, a_v = lax.fori_loop(
                0, _F, col_body, (zero, zero, zero, zero, m_v, a_v))
            var_o = var_o + (q_o - s_o * s_o * inv_f) * inv_fm1
            var_n = var_n + (q_n - s_n * s_n * inv_f) * inv_fm1
            return (var_o, var_n, m_v, a_v)

        return lax.fori_loop(0, _GROUPS, group_body, (var_o, var_n, m_v, a_v))

    var_o, var_n, m_v, a_v = lax.fori_loop(
        0, _N_CHUNKS, chunk_body, (zero, zero, zero, zero))

    out_v[0] = m_v
    out_v[1] = var_o
    out_v[2] = var_n
    out_v[3] = a_v
    pltpu.sync_copy(out_v, out_hbm.at[wid])


@jax.jit
def kernel(pred_noise, target_noise, content_original, content_from_noisy):
    co = content_original.reshape(_N_ROWS, _F)
    cf = content_from_noisy.reshape(_N_ROWS, _F)

    sc_out = _sc_content(co, cf)

    pred_spec = pl.BlockSpec((_PRED_B, _DIM, _N_PTS), lambda i: (i, 0, 0))
    cont_spec = pl.BlockSpec((_CONT_ROWS, _F), lambda i: (i, 0))

    tc_out = pl.pallas_call(
        _tc_kernel,
        grid=(_GRID,),
        in_specs=[pred_spec, pred_spec, cont_spec, cont_spec],
        out_specs=pl.BlockSpec(memory_space=pltpu.SMEM),
        out_shape=jax.ShapeDtypeStruct((5,), jnp.float32),
        scratch_shapes=[pltpu.SMEM((5,), jnp.float32)],
    )(jnp.swapaxes(pred_noise, 1, 2), jnp.swapaxes(target_noise, 1, 2),
      co, cf)

    sc_part = jnp.sum(sc_out, axis=(0, 2))
    s_diff = tc_out[0]
    s_mse = tc_out[1] + sc_part[0]
    s_var_o = tc_out[2] + sc_part[1]
    s_var_n = tc_out[3] + sc_part[2]
    s_abs = tc_out[4] + sc_part[3]

    diff_loss = s_diff / _N_DIFF
    mse_loss = s_mse / _N_CONTENT
    var_loss = (jnp.maximum(0.1 - s_var_o / _N_ROWS, 0.0)
                + jnp.maximum(0.1 - s_var_n / _N_ROWS, 0.0))
    act_loss = jnp.maximum(1.0 - s_abs / _N_CONTENT, 0.0) * 0.1
    content_loss = mse_loss + var_loss + act_loss
    total_loss = LAMBDA_DIFFUSION * diff_loss + LAMBDA_CONTENT * content_loss
    return jnp.stack([diff_loss, content_loss, total_loss])


# speedup vs baseline: 1.1688x; 1.1688x over previous
import functools

import jax
import jax.numpy as jnp
from jax import lax
from jax.experimental import pallas as pl
from jax.experimental.pallas import tpu as pltpu
from jax.experimental.pallas import tpu_sc as plsc

LAMBDA_DIFFUSION = 1.0
LAMBDA_CONTENT = 2.0

_B = 32
_N_PTS = 16384
_DIM = 3
_C = 512
_F = 1024

_N_DIFF = _B * _N_PTS * _DIM
_N_CONTENT = _B * _C * _F
_N_ROWS = _B * _C

_R_SC = 4096
_R_TC = _N_ROWS - _R_SC

_GRID = 16
_CONT_ROWS = _R_TC // _GRID
_PRED_B = _B // _GRID

_NW = 32
_ROWS_PER_W = _R_SC // _NW
_CHUNK = 32
_N_CHUNKS = _ROWS_PER_W // _CHUNK
_VECS = _F // 16


def _tc_kernel(pn_ref, tn_ref, co_ref, cf_ref, out_ref, acc_ref):
    i = pl.program_id(0)

    @pl.when(i == 0)
    def _init():
        for k in range(5):
            acc_ref[k] = 0.0

    d = pn_ref[...] - tn_ref[...]
    s_diff = jnp.sum(d * d)

    co = co_ref[...]
    cf = cf_ref[...]
    m = cf - co
    s_mse = jnp.sum(m * m)
    s_abs = jnp.sum(jnp.abs(co))

    inv_f = 1.0 / _F
    inv_fm1 = 1.0 / (_F - 1)
    sco = jnp.sum(co, axis=-1)
    sco2 = jnp.sum(co * co, axis=-1)
    s_var_o = jnp.sum((sco2 - sco * sco * inv_f) * inv_fm1)

    scf = jnp.sum(cf, axis=-1)
    scf2 = jnp.sum(cf * cf, axis=-1)
    s_var_n = jnp.sum((scf2 - scf * scf * inv_f) * inv_fm1)

    acc_ref[0] += s_diff
    acc_ref[1] += s_mse
    acc_ref[2] += s_var_o
    acc_ref[3] += s_var_n
    acc_ref[4] += s_abs

    @pl.when(i == _GRID - 1)
    def _finish():
        for k in range(5):
            out_ref[k] = acc_ref[k]


_sc_mesh = plsc.VectorSubcoreMesh(
    core_axis_name="c", subcore_axis_name="s", num_cores=2, num_subcores=16)


_GROUPS = _CHUNK // 16


_UNROLL = 8


@functools.partial(
    pl.kernel,
    out_type=jax.ShapeDtypeStruct((_NW, 4, 16), jnp.float32),
    mesh=_sc_mesh,
    scratch_types=[
        pltpu.VMEM((_CHUNK * _F,), jnp.float32),
        pltpu.VMEM((_CHUNK * _F,), jnp.float32),
        pltpu.VMEM((4, 16), jnp.float32),
    ],
    compiler_params=pltpu.CompilerParams(use_tc_tiling_on_sc=False, needs_layout_passes=False),
)
def _sc_content(co_hbm, cf_hbm, out_hbm, co_v, cf_v, out_v):
    wid = lax.axis_index("s") * 2 + lax.axis_index("c")
    base = (_R_TC + wid * _ROWS_PER_W) * _F
    zero = jnp.zeros((16,), jnp.float32)
    lane = lax.iota(jnp.int32, 16)
    inv_f = 1.0 / _F
    inv_fm1 = 1.0 / (_F - 1)

    def chunk_body(j, carry):
        var_o, var_n, m_v, a_v = carry
        e0 = base + j * (_CHUNK * _F)
        pltpu.sync_copy(co_hbm.at[pl.ds(e0, _CHUNK * _F)], co_v)
        pltpu.sync_copy(cf_hbm.at[pl.ds(e0, _CHUNK * _F)], cf_v)

        def group_body(g, carry2):
            var_o, var_n, m_v, a_v = carry2
            rowbase = (g * 16 + lane) * _F

            def col_body(c, c3):
                s_o, q_o, s_n, q_n, m_v, a_v = c3
                for u in range(_UNROLL):
                    idx = rowbase + (c * _UNROLL + u)
                    x = plsc.load_gather(co_v, [idx])
                    y = plsc.load_gather(cf_v, [idx])
                    s_o = s_o + x
                    q_o = q_o + x * x
                    s_n = s_n + y
                    q_n = q_n + y * y
                    d = y - x
                    m_v = m_v + d * d
                    a_v = a_v + jnp.abs(x)
                return (s_o, q_o, s_n, q_n, m_v, a_v)

            s_o, q_o, s_n, q_n, m_v, a_v = lax.fori_loop(
                0, _F // _UNROLL, col_body, (zero, zero, zero, zero, m_v, a_v))
            var_o = var_o + (q_o - s_o * s_o * inv_f) * inv_fm1
            var_n = var_n + (q_n - s_n * s_n * inv_f) * inv_fm1
            return (var_o, var_n, m_v, a_v)

        return lax.fori_loop(0, _GROUPS, group_body, (var_o, var_n, m_v, a_v))

    var_o, var_n, m_v, a_v = lax.fori_loop(
        0, _N_CHUNKS, chunk_body, (zero, zero, zero, zero))

    out_v[0] = m_v
    out_v[1] = var_o
    out_v[2] = var_n
    out_v[3] = a_v
    pltpu.sync_copy(out_v, out_hbm.at[wid])


@jax.jit
def kernel(pred_noise, target_noise, content_original, content_from_noisy):
    co = content_original.reshape(_N_ROWS, _F)
    cf = content_from_noisy.reshape(_N_ROWS, _F)

    sc_out = _sc_content(co.reshape(_N_CONTENT), cf.reshape(_N_CONTENT))

    pred_spec = pl.BlockSpec((_PRED_B, _DIM, _N_PTS), lambda i: (i, 0, 0))
    cont_spec = pl.BlockSpec((_CONT_ROWS, _F), lambda i: (i, 0))

    tc_out = pl.pallas_call(
        _tc_kernel,
        grid=(_GRID,),
        in_specs=[pred_spec, pred_spec, cont_spec, cont_spec],
        out_specs=pl.BlockSpec(memory_space=pltpu.SMEM),
        out_shape=jax.ShapeDtypeStruct((5,), jnp.float32),
        scratch_shapes=[pltpu.SMEM((5,), jnp.float32)],
    )(jnp.swapaxes(pred_noise, 1, 2), jnp.swapaxes(target_noise, 1, 2),
      co, cf)

    sc_part = jnp.sum(sc_out, axis=(0, 2))
    s_diff = tc_out[0]
    s_mse = tc_out[1] + sc_part[0]
    s_var_o = tc_out[2] + sc_part[1]
    s_var_n = tc_out[3] + sc_part[2]
    s_abs = tc_out[4] + sc_part[3]

    diff_loss = s_diff / _N_DIFF
    mse_loss = s_mse / _N_CONTENT
    var_loss = (jnp.maximum(0.1 - s_var_o / _N_ROWS, 0.0)
                + jnp.maximum(0.1 - s_var_n / _N_ROWS, 0.0))
    act_loss = jnp.maximum(1.0 - s_abs / _N_CONTENT, 0.0) * 0.1
    content_loss = mse_loss + var_loss + act_loss
    total_loss = LAMBDA_DIFFUSION * diff_loss + LAMBDA_CONTENT * content_loss
    return jnp.stack([diff_loss, content_loss, total_loss])


# speedup vs baseline: 3.4843x; 2.9811x over previous
import functools

import jax
import jax.numpy as jnp
from jax import lax
from jax.experimental import pallas as pl
from jax.experimental.pallas import tpu as pltpu
from jax.experimental.pallas import tpu_sc as plsc

LAMBDA_DIFFUSION = 1.0
LAMBDA_CONTENT = 2.0

_B = 32
_N_PTS = 16384
_DIM = 3
_C = 512
_F = 1024

_N_DIFF = _B * _N_PTS * _DIM
_N_CONTENT = _B * _C * _F
_N_ROWS = _B * _C

_R_SC = 5120
_R_TC = _N_ROWS - _R_SC

_GRID = 8
_CONT_ROWS = _R_TC // _GRID
_PRED_B = _B // _GRID

_NW = 32
_ROWS_PER_W = _R_SC // _NW
_CHUNK = 32
_N_CHUNKS = _ROWS_PER_W // _CHUNK
_VECS = _F // 16
_UNROLL = 8


def _tc_kernel(pn_ref, tn_ref, co_ref, cf_ref, out_ref, acc_ref):
    i = pl.program_id(0)

    @pl.when(i == 0)
    def _init():
        for k in range(5):
            acc_ref[k] = 0.0

    d = pn_ref[...] - tn_ref[...]
    s_diff = jnp.sum(d * d)

    co = co_ref[...]
    cf = cf_ref[...]
    m = cf - co
    s_mse = jnp.sum(m * m)
    s_abs = jnp.sum(jnp.abs(co))

    inv_f = 1.0 / _F
    inv_fm1 = 1.0 / (_F - 1)
    sco = jnp.sum(co, axis=-1)
    sco2 = jnp.sum(co * co, axis=-1)
    s_var_o = jnp.sum((sco2 - sco * sco * inv_f) * inv_fm1)

    scf = jnp.sum(cf, axis=-1)
    scf2 = jnp.sum(cf * cf, axis=-1)
    s_var_n = jnp.sum((scf2 - scf * scf * inv_f) * inv_fm1)

    acc_ref[0] += s_diff
    acc_ref[1] += s_mse
    acc_ref[2] += s_var_o
    acc_ref[3] += s_var_n
    acc_ref[4] += s_abs

    @pl.when(i == _GRID - 1)
    def _finish():
        for k in range(5):
            out_ref[k] = acc_ref[k]


_sc_mesh = plsc.VectorSubcoreMesh(
    core_axis_name="c", subcore_axis_name="s", num_cores=2, num_subcores=16)

_GDN = lax.GatherDimensionNumbers(
    offset_dims=(), collapsed_slice_dims=(0,), start_index_map=(0,))


def _shuffle(x, idx):
    return lax.gather(x, idx[:, None], _GDN, slice_sizes=(1,),
                      mode=lax.GatherScatterMode.PROMISE_IN_BOUNDS)


def _allsum(x, lane):
    for k in (1, 2, 4, 8):
        x = x + _shuffle(x, lane ^ k)
    return x


@functools.partial(
    pl.kernel,
    out_type=jax.ShapeDtypeStruct((_NW, 4, 16), jnp.float32),
    mesh=_sc_mesh,
    scratch_types=[
        pltpu.VMEM((_CHUNK, _F), jnp.float32),
        pltpu.VMEM((_CHUNK, _F), jnp.float32),
        pltpu.VMEM((4, 16), jnp.float32),
    ],
)
def _sc_content(co_hbm, cf_hbm, out_hbm, co_v, cf_v, out_v):
    wid = lax.axis_index("s") * 2 + lax.axis_index("c")
    base = _R_TC + wid * _ROWS_PER_W
    zero = jnp.zeros((16,), jnp.float32)
    lane = lax.iota(jnp.int32, 16)
    lane0 = lane == 0
    inv_f = 1.0 / _F
    inv_fm1 = 1.0 / (_F - 1)

    def chunk_body(j, carry):
        var_o, var_n, m_v, a_v = carry
        r0 = base + j * _CHUNK
        pltpu.sync_copy(co_hbm.at[pl.ds(r0, _CHUNK), :], co_v)
        pltpu.sync_copy(cf_hbm.at[pl.ds(r0, _CHUNK), :], cf_v)

        def row_body(r, carry2):
            var_o, var_n, m_v, a_v = carry2

            def col_body(c, c3):
                s0, q0, s1, q1, s2, q2, s3, q3, m_v, a_v = c3
                base_c = c * (_UNROLL * 16)
                for u in range(_UNROLL):
                    x = co_v[r, pl.ds(base_c + u * 16, 16)]
                    y = cf_v[r, pl.ds(base_c + u * 16, 16)]
                    if u % 2 == 0:
                        s0 = s0 + x
                        q0 = q0 + x * x
                        s1 = s1 + y
                        q1 = q1 + y * y
                    else:
                        s2 = s2 + x
                        q2 = q2 + x * x
                        s3 = s3 + y
                        q3 = q3 + y * y
                    d = y - x
                    m_v = m_v + d * d
                    a_v = a_v + jnp.abs(x)
                return (s0, q0, s1, q1, s2, q2, s3, q3, m_v, a_v)

            s0, q0, s1, q1, s2, q2, s3, q3, m_v, a_v = lax.fori_loop(
                0, _VECS // _UNROLL, col_body,
                (zero, zero, zero, zero, zero, zero, zero, zero, m_v, a_v))
            rs_o = _allsum(s0 + s2, lane)
            rq_o = _allsum(q0 + q2, lane)
            rs_n = _allsum(s1 + s3, lane)
            rq_n = _allsum(q1 + q3, lane)
            vo = (rq_o - rs_o * rs_o * inv_f) * inv_fm1
            vn = (rq_n - rs_n * rs_n * inv_f) * inv_fm1
            var_o = var_o + jnp.where(lane0, vo, 0.0)
            var_n = var_n + jnp.where(lane0, vn, 0.0)
            return (var_o, var_n, m_v, a_v)

        return lax.fori_loop(0, _CHUNK, row_body, (var_o, var_n, m_v, a_v))

    var_o, var_n, m_v, a_v = lax.fori_loop(
        0, _N_CHUNKS, chunk_body, (zero, zero, zero, zero))

    out_v[0] = m_v
    out_v[1] = var_o
    out_v[2] = var_n
    out_v[3] = a_v
    pltpu.sync_copy(out_v, out_hbm.at[wid])


@jax.jit
def kernel(pred_noise, target_noise, content_original, content_from_noisy):
    co = content_original.reshape(_N_ROWS, _F)
    cf = content_from_noisy.reshape(_N_ROWS, _F)

    sc_out = _sc_content(co, cf)

    pred_spec = pl.BlockSpec((_PRED_B, _DIM, _N_PTS), lambda i: (i, 0, 0))
    cont_spec = pl.BlockSpec((_CONT_ROWS, _F), lambda i: (i, 0))

    tc_out = pl.pallas_call(
        _tc_kernel,
        grid=(_GRID,),
        in_specs=[pred_spec, pred_spec, cont_spec, cont_spec],
        out_specs=pl.BlockSpec(memory_space=pltpu.SMEM),
        out_shape=jax.ShapeDtypeStruct((5,), jnp.float32),
        scratch_shapes=[pltpu.SMEM((5,), jnp.float32)],
    )(jnp.swapaxes(pred_noise, 1, 2), jnp.swapaxes(target_noise, 1, 2),
      co, cf)

    sc_part = jnp.sum(sc_out, axis=(0, 2))
    s_diff = tc_out[0]
    s_mse = tc_out[1] + sc_part[0]
    s_var_o = tc_out[2] + sc_part[1]
    s_var_n = tc_out[3] + sc_part[2]
    s_abs = tc_out[4] + sc_part[3]

    diff_loss = s_diff / _N_DIFF
    mse_loss = s_mse / _N_CONTENT
    var_loss = (jnp.maximum(0.1 - s_var_o / _N_ROWS, 0.0)
                + jnp.maximum(0.1 - s_var_n / _N_ROWS, 0.0))
    act_loss = jnp.maximum(1.0 - s_abs / _N_CONTENT, 0.0) * 0.1
    content_loss = mse_loss + var_loss + act_loss
    total_loss = LAMBDA_DIFFUSION * diff_loss + LAMBDA_CONTENT * content_loss
    return jnp.stack([diff_loss, content_loss, total_loss])


# speedup vs baseline: 3.5499x; 1.0188x over previous
import functools

import jax
import jax.numpy as jnp
from jax import lax
from jax.experimental import pallas as pl
from jax.experimental.pallas import tpu as pltpu
from jax.experimental.pallas import tpu_sc as plsc

LAMBDA_DIFFUSION = 1.0
LAMBDA_CONTENT = 2.0

_B = 32
_N_PTS = 16384
_DIM = 3
_C = 512
_F = 1024

_N_DIFF = _B * _N_PTS * _DIM
_N_CONTENT = _B * _C * _F
_N_ROWS = _B * _C

_R_SC = 3072
_R_TC = _N_ROWS - _R_SC

_GRID = 8
_CONT_ROWS = _R_TC // _GRID
_PRED_B = _B // _GRID

_NW = 32
_ROWS_PER_W = _R_SC // _NW
_CHUNK = 32
_N_CHUNKS = _ROWS_PER_W // _CHUNK
_VECS = _F // 16
_UNROLL = 8


def _tc_kernel(pn_ref, tn_ref, co_ref, cf_ref, out_ref, acc_ref):
    i = pl.program_id(0)

    @pl.when(i == 0)
    def _init():
        for k in range(5):
            acc_ref[k] = 0.0

    d = pn_ref[...] - tn_ref[...]
    s_diff = jnp.sum(d * d)

    co = co_ref[...]
    cf = cf_ref[...]
    m = cf - co
    s_mse = jnp.sum(m * m)
    s_abs = jnp.sum(jnp.abs(co))

    inv_f = 1.0 / _F
    inv_fm1 = 1.0 / (_F - 1)
    sco = jnp.sum(co, axis=-1)
    sco2 = jnp.sum(co * co, axis=-1)
    s_var_o = jnp.sum((sco2 - sco * sco * inv_f) * inv_fm1)

    scf = jnp.sum(cf, axis=-1)
    scf2 = jnp.sum(cf * cf, axis=-1)
    s_var_n = jnp.sum((scf2 - scf * scf * inv_f) * inv_fm1)

    acc_ref[0] += s_diff
    acc_ref[1] += s_mse
    acc_ref[2] += s_var_o
    acc_ref[3] += s_var_n
    acc_ref[4] += s_abs

    @pl.when(i == _GRID - 1)
    def _finish():
        for k in range(5):
            out_ref[k] = acc_ref[k]


_sc_mesh = plsc.VectorSubcoreMesh(
    core_axis_name="c", subcore_axis_name="s", num_cores=2, num_subcores=16)

_GDN = lax.GatherDimensionNumbers(
    offset_dims=(), collapsed_slice_dims=(0,), start_index_map=(0,))


def _shuffle(x, idx):
    return lax.gather(x, idx[:, None], _GDN, slice_sizes=(1,),
                      mode=lax.GatherScatterMode.PROMISE_IN_BOUNDS)


def _allsum(x, lane):
    for k in (1, 2, 4, 8):
        x = x + _shuffle(x, lane ^ k)
    return x


@functools.partial(
    pl.kernel,
    out_type=jax.ShapeDtypeStruct((_NW, 4, 16), jnp.float32),
    mesh=_sc_mesh,
    scratch_types=[
        pltpu.VMEM((_CHUNK, _F), jnp.float32),
        pltpu.VMEM((_CHUNK, _F), jnp.float32),
        pltpu.VMEM((4, 16), jnp.float32),
    ],
)
def _sc_content(co_hbm, cf_hbm, out_hbm, co_v, cf_v, out_v):
    wid = lax.axis_index("s") * 2 + lax.axis_index("c")
    base = _R_TC + wid * _ROWS_PER_W
    zero = jnp.zeros((16,), jnp.float32)
    lane = lax.iota(jnp.int32, 16)
    lane0 = lane == 0
    inv_f = 1.0 / _F
    inv_fm1 = 1.0 / (_F - 1)

    def chunk_body(j, carry):
        var_o, var_n, m_v, a_v = carry
        r0 = base + j * _CHUNK
        pltpu.sync_copy(co_hbm.at[pl.ds(r0, _CHUNK), :], co_v)
        pltpu.sync_copy(cf_hbm.at[pl.ds(r0, _CHUNK), :], cf_v)

        def row_body(r, carry2):
            var_o, var_n, m_v, a_v = carry2

            def col_body(c, c3):
                s0, q0, s1, q1, s2, q2, s3, q3, m_v, a_v = c3
                base_c = c * (_UNROLL * 16)
                for u in range(_UNROLL):
                    x = co_v[r, pl.ds(base_c + u * 16, 16)]
                    y = cf_v[r, pl.ds(base_c + u * 16, 16)]
                    if u % 2 == 0:
                        s0 = s0 + x
                        q0 = q0 + x * x
                        s1 = s1 + y
                        q1 = q1 + y * y
                    else:
                        s2 = s2 + x
                        q2 = q2 + x * x
                        s3 = s3 + y
                        q3 = q3 + y * y
                    d = y - x
                    m_v = m_v + d * d
                    a_v = a_v + jnp.abs(x)
                return (s0, q0, s1, q1, s2, q2, s3, q3, m_v, a_v)

            s0, q0, s1, q1, s2, q2, s3, q3, m_v, a_v = lax.fori_loop(
                0, _VECS // _UNROLL, col_body,
                (zero, zero, zero, zero, zero, zero, zero, zero, m_v, a_v))
            rs_o = _allsum(s0 + s2, lane)
            rq_o = _allsum(q0 + q2, lane)
            rs_n = _allsum(s1 + s3, lane)
            rq_n = _allsum(q1 + q3, lane)
            vo = (rq_o - rs_o * rs_o * inv_f) * inv_fm1
            vn = (rq_n - rs_n * rs_n * inv_f) * inv_fm1
            var_o = var_o + jnp.where(lane0, vo, 0.0)
            var_n = var_n + jnp.where(lane0, vn, 0.0)
            return (var_o, var_n, m_v, a_v)

        return lax.fori_loop(0, _CHUNK, row_body, (var_o, var_n, m_v, a_v))

    var_o, var_n, m_v, a_v = lax.fori_loop(
        0, _N_CHUNKS, chunk_body, (zero, zero, zero, zero))

    out_v[0] = m_v
    out_v[1] = var_o
    out_v[2] = var_n
    out_v[3] = a_v
    pltpu.sync_copy(out_v, out_hbm.at[wid])


@jax.jit
def kernel(pred_noise, target_noise, content_original, content_from_noisy):
    co = content_original.reshape(_N_ROWS, _F)
    cf = content_from_noisy.reshape(_N_ROWS, _F)

    sc_out = _sc_content(co, cf)

    pred_spec = pl.BlockSpec((_PRED_B, _DIM, _N_PTS), lambda i: (i, 0, 0))
    cont_spec = pl.BlockSpec((_CONT_ROWS, _F), lambda i: (i, 0))

    tc_out = pl.pallas_call(
        _tc_kernel,
        grid=(_GRID,),
        in_specs=[pred_spec, pred_spec, cont_spec, cont_spec],
        out_specs=pl.BlockSpec(memory_space=pltpu.SMEM),
        out_shape=jax.ShapeDtypeStruct((5,), jnp.float32),
        scratch_shapes=[pltpu.SMEM((5,), jnp.float32)],
    )(jnp.swapaxes(pred_noise, 1, 2), jnp.swapaxes(target_noise, 1, 2),
      co, cf)

    sc_part = jnp.sum(sc_out, axis=(0, 2))
    s_diff = tc_out[0]
    s_mse = tc_out[1] + sc_part[0]
    s_var_o = tc_out[2] + sc_part[1]
    s_var_n = tc_out[3] + sc_part[2]
    s_abs = tc_out[4] + sc_part[3]

    diff_loss = s_diff / _N_DIFF
    mse_loss = s_mse / _N_CONTENT
    var_loss = (jnp.maximum(0.1 - s_var_o / _N_ROWS, 0.0)
                + jnp.maximum(0.1 - s_var_n / _N_ROWS, 0.0))
    act_loss = jnp.maximum(1.0 - s_abs / _N_CONTENT, 0.0) * 0.1
    content_loss = mse_loss + var_loss + act_loss
    total_loss = LAMBDA_DIFFUSION * diff_loss + LAMBDA_CONTENT * content_loss
    return jnp.stack([diff_loss, content_loss, total_loss])


# speedup vs baseline: 3.6514x; 1.0286x over previous
import functools

import jax
import jax.numpy as jnp
from jax import lax
from jax.experimental import pallas as pl
from jax.experimental.pallas import tpu as pltpu
from jax.experimental.pallas import tpu_sc as plsc

LAMBDA_DIFFUSION = 1.0
LAMBDA_CONTENT = 2.0

_B = 32
_N_PTS = 16384
_DIM = 3
_C = 512
_F = 1024

_N_DIFF = _B * _N_PTS * _DIM
_N_CONTENT = _B * _C * _F
_N_ROWS = _B * _C

_R_SC = 4096
_R_TC = _N_ROWS - _R_SC

_GRID = 8
_CONT_ROWS = _R_TC // _GRID
_PRED_B = _B // _GRID

_NW = 32
_ROWS_PER_W = _R_SC // _NW
_CHUNK = 32
_N_CHUNKS = _ROWS_PER_W // _CHUNK
_VECS = _F // 16
_UNROLL = 8


def _tc_kernel(pn_ref, tn_ref, co_ref, cf_ref, out_ref, acc_ref):
    i = pl.program_id(0)

    @pl.when(i == 0)
    def _init():
        for k in range(5):
            acc_ref[k] = 0.0

    d = pn_ref[...] - tn_ref[...]
    s_diff = jnp.sum(d * d)

    co = co_ref[...]
    cf = cf_ref[...]
    m = cf - co
    s_mse = jnp.sum(m * m)
    s_abs = jnp.sum(jnp.abs(co))

    inv_f = 1.0 / _F
    inv_fm1 = 1.0 / (_F - 1)
    sco = jnp.sum(co, axis=-1)
    sco2 = jnp.sum(co * co, axis=-1)
    s_var_o = jnp.sum((sco2 - sco * sco * inv_f) * inv_fm1)

    scf = jnp.sum(cf, axis=-1)
    scf2 = jnp.sum(cf * cf, axis=-1)
    s_var_n = jnp.sum((scf2 - scf * scf * inv_f) * inv_fm1)

    acc_ref[0] += s_diff
    acc_ref[1] += s_mse
    acc_ref[2] += s_var_o
    acc_ref[3] += s_var_n
    acc_ref[4] += s_abs

    @pl.when(i == _GRID - 1)
    def _finish():
        for k in range(5):
            out_ref[k] = acc_ref[k]


_sc_mesh = plsc.VectorSubcoreMesh(
    core_axis_name="c", subcore_axis_name="s", num_cores=2, num_subcores=16)

_GDN = lax.GatherDimensionNumbers(
    offset_dims=(), collapsed_slice_dims=(0,), start_index_map=(0,))


def _shuffle(x, idx):
    return lax.gather(x, idx[:, None], _GDN, slice_sizes=(1,),
                      mode=lax.GatherScatterMode.PROMISE_IN_BOUNDS)


def _allsum(x, lane):
    for k in (1, 2, 4, 8):
        x = x + _shuffle(x, lane ^ k)
    return x


@functools.partial(
    pl.kernel,
    out_type=jax.ShapeDtypeStruct((_NW, 4, 16), jnp.float32),
    mesh=_sc_mesh,
    scratch_types=[
        pltpu.VMEM((_CHUNK, _F), jnp.float32),
        pltpu.VMEM((_CHUNK, _F), jnp.float32),
        pltpu.VMEM((4, 16), jnp.float32),
    ],
)
def _sc_content(co_hbm, cf_hbm, out_hbm, co_v, cf_v, out_v):
    wid = lax.axis_index("s") * 2 + lax.axis_index("c")
    base = _R_TC + wid * _ROWS_PER_W
    zero = jnp.zeros((16,), jnp.float32)
    lane = lax.iota(jnp.int32, 16)
    lane0 = lane == 0
    inv_f = 1.0 / _F
    inv_fm1 = 1.0 / (_F - 1)

    def chunk_body(j, carry):
        var_o, var_n, m_v, a_v = carry
        r0 = base + j * _CHUNK
        b = r0 // _C
        c0 = r0 % _C
        pltpu.sync_copy(co_hbm.at[b, pl.ds(c0, _CHUNK), :], co_v)
        pltpu.sync_copy(cf_hbm.at[b, pl.ds(c0, _CHUNK), :], cf_v)

        def row_body(r, carry2):
            var_o, var_n, m_v, a_v = carry2

            def col_body(c, c3):
                s0, q0, s1, q1, s2, q2, s3, q3, m_v, a_v = c3
                base_c = c * (_UNROLL * 16)
                for u in range(_UNROLL):
                    x = co_v[r, pl.ds(base_c + u * 16, 16)]
                    y = cf_v[r, pl.ds(base_c + u * 16, 16)]
                    if u % 2 == 0:
                        s0 = s0 + x
                        q0 = q0 + x * x
                        s1 = s1 + y
                        q1 = q1 + y * y
                    else:
                        s2 = s2 + x
                        q2 = q2 + x * x
                        s3 = s3 + y
                        q3 = q3 + y * y
                    d = y - x
                    m_v = m_v + d * d
                    a_v = a_v + jnp.abs(x)
                return (s0, q0, s1, q1, s2, q2, s3, q3, m_v, a_v)

            s0, q0, s1, q1, s2, q2, s3, q3, m_v, a_v = lax.fori_loop(
                0, _VECS // _UNROLL, col_body,
                (zero, zero, zero, zero, zero, zero, zero, zero, m_v, a_v))
            rs_o = _allsum(s0 + s2, lane)
            rq_o = _allsum(q0 + q2, lane)
            rs_n = _allsum(s1 + s3, lane)
            rq_n = _allsum(q1 + q3, lane)
            vo = (rq_o - rs_o * rs_o * inv_f) * inv_fm1
            vn = (rq_n - rs_n * rs_n * inv_f) * inv_fm1
            var_o = var_o + jnp.where(lane0, vo, 0.0)
            var_n = var_n + jnp.where(lane0, vn, 0.0)
            return (var_o, var_n, m_v, a_v)

        return lax.fori_loop(0, _CHUNK, row_body, (var_o, var_n, m_v, a_v))

    var_o, var_n, m_v, a_v = lax.fori_loop(
        0, _N_CHUNKS, chunk_body, (zero, zero, zero, zero))

    out_v[0] = m_v
    out_v[1] = var_o
    out_v[2] = var_n
    out_v[3] = a_v
    pltpu.sync_copy(out_v, out_hbm.at[wid])


@jax.jit
def kernel(pred_noise, target_noise, content_original, content_from_noisy):
    co = content_original.reshape(_N_ROWS, _F)
    cf = content_from_noisy.reshape(_N_ROWS, _F)

    sc_out = _sc_content(content_original, content_from_noisy)

    pred_spec = pl.BlockSpec((_PRED_B, _DIM, _N_PTS), lambda i: (i, 0, 0))
    cont_spec = pl.BlockSpec((_CONT_ROWS, _F), lambda i: (i, 0))

    tc_out = pl.pallas_call(
        _tc_kernel,
        grid=(_GRID,),
        in_specs=[pred_spec, pred_spec, cont_spec, cont_spec],
        out_specs=pl.BlockSpec(memory_space=pltpu.SMEM),
        out_shape=jax.ShapeDtypeStruct((5,), jnp.float32),
        scratch_shapes=[pltpu.SMEM((5,), jnp.float32)],
    )(jnp.swapaxes(pred_noise, 1, 2), jnp.swapaxes(target_noise, 1, 2),
      co, cf)

    sc_part = jnp.sum(sc_out, axis=(0, 2))
    s_diff = tc_out[0]
    s_mse = tc_out[1] + sc_part[0]
    s_var_o = tc_out[2] + sc_part[1]
    s_var_n = tc_out[3] + sc_part[2]
    s_abs = tc_out[4] + sc_part[3]

    diff_loss = s_diff / _N_DIFF
    mse_loss = s_mse / _N_CONTENT
    var_loss = (jnp.maximum(0.1 - s_var_o / _N_ROWS, 0.0)
                + jnp.maximum(0.1 - s_var_n / _N_ROWS, 0.0))
    act_loss = jnp.maximum(1.0 - s_abs / _N_CONTENT, 0.0) * 0.1
    content_loss = mse_loss + var_loss + act_loss
    total_loss = LAMBDA_DIFFUSION * diff_loss + LAMBDA_CONTENT * content_loss
    return jnp.stack([diff_loss, content_loss, total_loss])


# speedup vs baseline: 3.9089x; 1.0705x over previous
import jax
import jax.numpy as jnp
from jax.experimental import pallas as pl
from jax.experimental.pallas import tpu as pltpu

LAMBDA_DIFFUSION = 1.0
LAMBDA_CONTENT = 2.0

_B = 32
_N_PTS = 16384
_DIM = 3
_C = 512
_F = 1024

_N_DIFF = _B * _N_PTS * _DIM
_N_CONTENT = _B * _C * _F
_N_ROWS = _B * _C

_GRID = 8
_PRED_ROWS = _N_DIFF // 128 // _GRID
_CONT_ROWS = (_B * _C) // _GRID


def _loss_kernel(d_ref, co_ref, cf_ref, out_ref, acc_ref):
    i = pl.program_id(0)

    @pl.when(i == 0)
    def _init():
        for k in range(5):
            acc_ref[k] = 0.0

    d = d_ref[...]
    s_diff = jnp.sum(d * d)

    co = co_ref[...]
    cf = cf_ref[...]
    m = cf - co
    s_mse = jnp.sum(m * m)
    s_abs = jnp.sum(jnp.abs(co))

    inv_f = 1.0 / _F
    inv_fm1 = 1.0 / (_F - 1)
    sco = jnp.sum(co, axis=-1)
    sco2 = jnp.sum(co * co, axis=-1)
    var_o = (sco2 - sco * sco * inv_f) * inv_fm1
    s_var_o = jnp.sum(var_o)

    scf = jnp.sum(cf, axis=-1)
    scf2 = jnp.sum(cf * cf, axis=-1)
    var_n = (scf2 - scf * scf * inv_f) * inv_fm1
    s_var_n = jnp.sum(var_n)

    acc_ref[0] += s_diff
    acc_ref[1] += s_mse
    acc_ref[2] += s_var_o
    acc_ref[3] += s_var_n
    acc_ref[4] += s_abs

    @pl.when(i == _GRID - 1)
    def _finish():
        diff_loss = acc_ref[0] / _N_DIFF
        mse_loss = acc_ref[1] / _N_CONTENT
        var_o_mean = acc_ref[2] / _N_ROWS
        var_n_mean = acc_ref[3] / _N_ROWS
        var_loss = (jnp.maximum(0.1 - var_o_mean, 0.0)
                    + jnp.maximum(0.1 - var_n_mean, 0.0))
        act_loss = jnp.maximum(1.0 - acc_ref[4] / _N_CONTENT, 0.0) * 0.1
        content_loss = mse_loss + var_loss + act_loss
        total_loss = LAMBDA_DIFFUSION * diff_loss + LAMBDA_CONTENT * content_loss
        out_ref[0] = diff_loss
        out_ref[1] = content_loss
        out_ref[2] = total_loss


@jax.jit
def kernel(pred_noise, target_noise, content_original, content_from_noisy):
    co = content_original.reshape(_B * _C, _F)
    cf = content_from_noisy.reshape(_B * _C, _F)

    pred_spec = pl.BlockSpec((4, _DIM, _N_PTS), lambda i: (i, 0, 0))
    cont_spec = pl.BlockSpec((_CONT_ROWS, _F), lambda i: (i, 0))

    out = pl.pallas_call(
        _loss_kernel,
        grid=(_GRID,),
        in_specs=[pred_spec, cont_spec, cont_spec],
        out_specs=pl.BlockSpec(memory_space=pltpu.SMEM),
        out_shape=jax.ShapeDtypeStruct((3,), jnp.float32),
        scratch_shapes=[pltpu.SMEM((5,), jnp.float32)],
    )(jnp.swapaxes(pred_noise, 1, 2) - jnp.swapaxes(target_noise, 1, 2), co, cf)
    return out


# speedup vs baseline: 4.5770x; 1.1709x over previous
import jax
import jax.numpy as jnp
from jax.experimental import pallas as pl
from jax.experimental.pallas import tpu as pltpu

LAMBDA_DIFFUSION = 1.0
LAMBDA_CONTENT = 2.0

_B = 32
_N_PTS = 16384
_DIM = 3
_C = 512
_F = 1024

_N_DIFF = _B * _N_PTS * _DIM
_N_CONTENT = _B * _C * _F
_N_ROWS = _B * _C

_GRID = 8
_PRED_ROWS = _N_DIFF // 128 // _GRID
_CONT_ROWS = (_B * _C) // _GRID


def _loss_kernel(pn_ref, tn_ref, co_ref, cf_ref, out_ref, acc_ref):
    i = pl.program_id(0)

    @pl.when(i == 0)
    def _init():
        for k in range(5):
            acc_ref[k] = 0.0

    d = pn_ref[...] - tn_ref[...]
    s_diff = jnp.sum(d * d)

    co = co_ref[...]
    cf = cf_ref[...]
    m = cf - co
    s_mse = jnp.sum(m * m)
    s_abs = jnp.sum(jnp.abs(co))

    inv_f = 1.0 / _F
    inv_fm1 = 1.0 / (_F - 1)
    sco = jnp.sum(co, axis=-1)
    sco2 = jnp.sum(co * co, axis=-1)
    var_o = (sco2 - sco * sco * inv_f) * inv_fm1
    s_var_o = jnp.sum(var_o)

    scf = jnp.sum(cf, axis=-1)
    scf2 = jnp.sum(cf * cf, axis=-1)
    var_n = (scf2 - scf * scf * inv_f) * inv_fm1
    s_var_n = jnp.sum(var_n)

    acc_ref[0] += s_diff
    acc_ref[1] += s_mse
    acc_ref[2] += s_var_o
    acc_ref[3] += s_var_n
    acc_ref[4] += s_abs

    @pl.when(i == _GRID - 1)
    def _finish():
        diff_loss = acc_ref[0] / _N_DIFF
        mse_loss = acc_ref[1] / _N_CONTENT
        var_o_mean = acc_ref[2] / _N_ROWS
        var_n_mean = acc_ref[3] / _N_ROWS
        var_loss = (jnp.maximum(0.1 - var_o_mean, 0.0)
                    + jnp.maximum(0.1 - var_n_mean, 0.0))
        act_loss = jnp.maximum(1.0 - acc_ref[4] / _N_CONTENT, 0.0) * 0.1
        content_loss = mse_loss + var_loss + act_loss
        total_loss = LAMBDA_DIFFUSION * diff_loss + LAMBDA_CONTENT * content_loss
        out_ref[0] = diff_loss
        out_ref[1] = content_loss
        out_ref[2] = total_loss


@jax.jit
def kernel(pred_noise, target_noise, content_original, content_from_noisy):
    co = content_original.reshape(_B * _C, _F)
    cf = content_from_noisy.reshape(_B * _C, _F)

    pred_spec = pl.BlockSpec((4, _DIM, _N_PTS), lambda i: (i, 0, 0))
    cont_spec = pl.BlockSpec((_CONT_ROWS, _F), lambda i: (i, 0))

    out = pl.pallas_call(
        _loss_kernel,
        grid=(_GRID,),
        in_specs=[pred_spec, pred_spec, cont_spec, cont_spec],
        out_specs=pl.BlockSpec(memory_space=pltpu.SMEM),
        out_shape=jax.ShapeDtypeStruct((3,), jnp.float32),
        scratch_shapes=[pltpu.SMEM((5,), jnp.float32)],
    )(jnp.swapaxes(pred_noise, 1, 2), jnp.swapaxes(target_noise, 1, 2), co, cf)
    return out
